# baseline (speedup 1.0000x reference)
import functools

import jax
import jax.numpy as jnp
from jax import lax
from jax.experimental import pallas as pl
from jax.experimental.pallas import tpu as pltpu
from jax.experimental.pallas import tpu_sc as plsc

_POOL = 1024
_LEN = 8
_DIM = 768
_K = 4
_B = 4
_HALF = _LEN * _DIM // 2

_NOISE_CACHE = []


def _gumbel_noise():
    if not _NOISE_CACHE:
        with jax.ensure_compile_time_eval():
            gkey = jax.random.key(42)
            outs = []
            for _ in range(_K):
                gkey, sub = jax.random.split(gkey)
                u = jax.random.uniform(sub, (_B, _POOL), minval=1e-20, maxval=1.0)
                g = -jnp.log(-jnp.log(u) + 1e-20)
                outs.append(jnp.pad(g, ((0, 8 - _B), (0, 0))))
            _NOISE_CACHE.append(jnp.concatenate(outs, axis=0))
    return _NOISE_CACHE[0]


def _select_body(pk_ref, q_ref, g_ref, out_ref):
    pk = pk_ref[...]
    pk = pk * lax.rsqrt(jnp.maximum(jnp.sum(pk * pk, axis=1, keepdims=True), 1e-12))
    qv = q_ref[...]
    qv = qv * lax.rsqrt(jnp.maximum(jnp.sum(qv * qv, axis=1, keepdims=True), 1e-12))
    sim = lax.dot_general(qv, pk, (((1,), (1,)), ((), ())),
                          preferred_element_type=jnp.float32)
    iota = lax.broadcasted_iota(jnp.int32, (_B, _POOL), 1)
    cur = sim
    cols = []
    for r in range(_K):
        logits = cur + g_ref[pl.ds(8 * r, _B), :]
        m = jnp.max(logits, axis=1, keepdims=True)
        idx = jnp.min(jnp.where(logits == m, iota, jnp.int32(2 ** 30)),
                      axis=1, keepdims=True)
        cols.append(idx)
        cur = jnp.where(iota == idx, cur - 1000.0, cur)
    sel = jnp.concatenate(cols, axis=0)

    pio = lax.broadcasted_iota(jnp.int32, (_K * _B, 128), 0)
    qio = lax.broadcasted_iota(jnp.int32, (_K * _B, 128), 1)
    perm = ((qio >> 1) & 3) * 4 + (qio >> 3)
    pmat = jnp.where((pio == perm) & (qio < 32), 1.0, 0.0)
    hi = (sel >> 2).astype(jnp.float32)
    lo = (sel & 3).astype(jnp.float32)
    dims = (((0,), (0,)), ((), ()))
    e_hi = lax.dot_general(hi, pmat, dims, preferred_element_type=jnp.float32)
    e_lo = lax.dot_general(lo, pmat, dims, preferred_element_type=jnp.float32)
    lane = lax.broadcasted_iota(jnp.int32, (1, 128), 1)
    e = (8.0 * e_hi + 2.0 * e_lo + (lane & 1).astype(jnp.float32)).astype(jnp.int32)
    out_ref[...] = jnp.broadcast_to(e, (8, 128))


def _gather(emap, table):
    mesh = plsc.VectorSubcoreMesh(core_axis_name="c", subcore_axis_name="s")

    @functools.partial(
        pl.kernel,
        mesh=mesh,
        out_type=jax.ShapeDtypeStruct((2 * _K * _B, _HALF), jnp.float32),
        scratch_types=[
            pltpu.VMEM((16,), jnp.int32),
            pltpu.VMEM((16, _HALF), jnp.float32),
            pltpu.SemaphoreType.DMA,
        ],
    )
    def k(emap_hbm, tab_hbm, out_hbm, idx_v, rows_v, sem):
        c = lax.axis_index("c")
        s = lax.axis_index("s")

        @pl.when(s == 0)
        def _():
            pltpu.sync_copy(emap_hbm.at[0, pl.ds(c * 16, 16)], idx_v)
            pltpu.async_copy(tab_hbm.at[idx_v], rows_v, sem).wait()
            pltpu.sync_copy(rows_v, out_hbm.at[pl.ds(c * 16, 16), :])

    return k(emap, table)


def kernel(x_embed, cls_features, prompt, prompt_key):
    del x_embed
    emap = pl.pallas_call(
        _select_body,
        out_shape=jax.ShapeDtypeStruct((8, 128), jnp.int32),
    )(prompt_key, cls_features, _gumbel_noise())
    table = prompt.reshape(2 * _POOL, _HALF)
    rows = _gather(emap, table)
    return rows.reshape(_B, _K * _LEN, _DIM)


# speedup vs baseline: 2.4728x; 2.4728x over previous
import functools

import jax
import jax.numpy as jnp
from jax import lax
from jax.experimental import pallas as pl
from jax.experimental.pallas import tpu as pltpu
from jax.experimental.pallas import tpu_sc as plsc

_POOL = 1024
_LEN = 8
_DIM = 768
_K = 4
_B = 4

_NOISE_CACHE = []


def _gumbel_noise():
    if not _NOISE_CACHE:
        with jax.ensure_compile_time_eval():
            gkey = jax.random.key(42)
            outs = []
            for _ in range(_K):
                gkey, sub = jax.random.split(gkey)
                u = jax.random.uniform(sub, (_B, _POOL), minval=1e-20, maxval=1.0)
                g = -jnp.log(-jnp.log(u) + 1e-20)
                outs.append(jnp.pad(g, ((0, 8 - _B), (0, 0))))
            _NOISE_CACHE.append(jnp.concatenate(outs, axis=0))
    return _NOISE_CACHE[0]


def _select_body(pk_ref, q_ref, g_ref, out_ref):
    pk = pk_ref[...]
    pk = pk * lax.rsqrt(jnp.maximum(jnp.sum(pk * pk, axis=1, keepdims=True), 1e-12))
    qv = q_ref[...]
    qv = qv * lax.rsqrt(jnp.maximum(jnp.sum(qv * qv, axis=1, keepdims=True), 1e-12))
    sim = lax.dot_general(qv, pk, (((1,), (1,)), ((), ())),
                          preferred_element_type=jnp.float32)
    iota = lax.broadcasted_iota(jnp.int32, (_B, _POOL), 1)
    cur = sim
    cols = []
    for r in range(_K):
        logits = cur + g_ref[pl.ds(8 * r, _B), :]
        m = jnp.max(logits, axis=1, keepdims=True)
        idx = jnp.min(jnp.where(logits == m, iota, jnp.int32(2 ** 30)),
                      axis=1, keepdims=True)
        cols.append(idx)
        cur = jnp.where(iota == idx, cur - 1000.0, cur)
    sel = jnp.concatenate(cols, axis=0)
    out_ref[...] = jnp.broadcast_to(sel, (_K * _B, 128))


def _gather(emap, prompt):
    mesh = plsc.VectorSubcoreMesh(core_axis_name="c", subcore_axis_name="s")

    @functools.partial(
        pl.kernel,
        mesh=mesh,
        out_type=jax.ShapeDtypeStruct((_K * _B, _LEN, _DIM), jnp.float32),
        scratch_types=[
            pltpu.VMEM((16,), jnp.int32),
            pltpu.VMEM((1, _LEN, _DIM), jnp.float32),
            pltpu.SemaphoreType.DMA,
        ],
    )
    def k(emap_hbm, prompt_hbm, out_hbm, idx_v, row_v, sem):
        w = lax.axis_index("s") * 2 + lax.axis_index("c")

        @pl.when(w < _K * _B)
        def _():
            p = (w & 3) * 4 + (w >> 2)
            pltpu.sync_copy(emap_hbm.at[p, pl.ds(0, 16)], idx_v)
            pltpu.async_copy(prompt_hbm.at[idx_v.at[pl.ds(0, 1)]], row_v, sem).wait()
            pltpu.sync_copy(row_v, out_hbm.at[pl.ds(w, 1), :, :])

    return k(emap, prompt)


def kernel(x_embed, cls_features, prompt, prompt_key):
    del x_embed
    emap = pl.pallas_call(
        _select_body,
        out_shape=jax.ShapeDtypeStruct((_K * _B, 128), jnp.int32),
    )(prompt_key, cls_features, _gumbel_noise())
    rows = _gather(emap, prompt)
    return rows.reshape(_B, _K * _LEN, _DIM)


# speedup vs baseline: 2.6509x; 1.0720x over previous
import functools

import jax
import jax.numpy as jnp
from jax import lax
from jax.experimental import pallas as pl
from jax.experimental.pallas import tpu as pltpu
from jax.experimental.pallas import tpu_sc as plsc

_POOL = 1024
_LEN = 8
_DIM = 768
_K = 4
_B = 4

_NOISE_CACHE = []


def _gumbel_noise():
    if not _NOISE_CACHE:
        with jax.ensure_compile_time_eval():
            gkey = jax.random.key(42)
            outs = []
            for _ in range(_K):
                gkey, sub = jax.random.split(gkey)
                u = jax.random.uniform(sub, (_B, _POOL), minval=1e-20, maxval=1.0)
                g = -jnp.log(-jnp.log(u) + 1e-20)
                outs.append(jnp.pad(g, ((0, 8 - _B), (0, 0))))
            _NOISE_CACHE.append(jnp.concatenate(outs, axis=0))
    return _NOISE_CACHE[0]


def _select_body(pk_ref, q_ref, g_ref, out_ref):
    pk = pk_ref[...]
    pk = pk * lax.rsqrt(jnp.maximum(jnp.sum(pk * pk, axis=1, keepdims=True), 1e-12))
    qv = q_ref[...]
    qv = qv * lax.rsqrt(jnp.maximum(jnp.sum(qv * qv, axis=1, keepdims=True), 1e-12))
    sim = lax.dot_general(qv, pk, (((1,), (1,)), ((), ())),
                          preferred_element_type=jnp.float32)
    iota = lax.broadcasted_iota(jnp.int32, (_B, _POOL), 1)
    cur = sim
    cols = []
    for r in range(_K):
        logits = cur + g_ref[pl.ds(8 * r, _B), :]
        m = jnp.max(logits, axis=1, keepdims=True)
        idx = jnp.min(jnp.where(logits == m, iota, jnp.int32(2 ** 30)),
                      axis=1, keepdims=True)
        cols.append(idx)
        cur = jnp.where(iota == idx, cur - 1000.0, cur)
    sel = jnp.concatenate(cols, axis=0)
    out_ref[...] = jnp.broadcast_to(sel, (_K * _B, 128))


def _gather(emap, prompt):
    mesh = plsc.VectorSubcoreMesh(core_axis_name="c", subcore_axis_name="s",
                                  num_cores=1)

    @functools.partial(
        pl.kernel,
        mesh=mesh,
        out_type=jax.ShapeDtypeStruct((_K * _B, _LEN, _DIM), jnp.float32),
        scratch_types=[
            pltpu.VMEM((16,), jnp.int32),
            pltpu.VMEM((1, _LEN, _DIM), jnp.float32),
            pltpu.SemaphoreType.DMA,
        ],
    )
    def k(emap_hbm, prompt_hbm, out_hbm, idx_v, row_v, sem):
        w = lax.axis_index("s")

        p = (w & 3) * 4 + (w >> 2)
        pltpu.sync_copy(emap_hbm.at[p, pl.ds(0, 16)], idx_v)
        pltpu.async_copy(prompt_hbm.at[idx_v.at[pl.ds(0, 1)]], row_v, sem).wait()
        pltpu.sync_copy(row_v, out_hbm.at[pl.ds(w, 1), :, :])

    return k(emap, prompt)


def kernel(x_embed, cls_features, prompt, prompt_key):
    del x_embed
    emap = pl.pallas_call(
        _select_body,
        out_shape=jax.ShapeDtypeStruct((_K * _B, 128), jnp.int32),
    )(prompt_key, cls_features, _gumbel_noise())
    rows = _gather(emap, prompt)
    return rows.reshape(_B, _K * _LEN, _DIM)
